# baseline (speedup 1.0000x reference)
import functools

import jax
import jax.numpy as jnp
from jax import lax
from jax.experimental import pallas as pl
from jax.experimental.pallas import tpu as pltpu
from jax.experimental.pallas import tpu_sc as plsc

NN = 10000
EE = 320000
DD = 128
HH = 64
OO = 32

NC = 2
NS = 16
NW = NC * NS
CHUNK = 128
KCH = 80
EPAD = NW * KCH * CHUNK
NACC = 10112
RPT = NACC // NS


def _seg_sum_body(f, nb, y_hbm, src_hbm, dst_hbm, zrow_hbm, *refs):
    acc_out, sidx_v, didx_v, rows_v, acc_sh, gsem, ssem, zsem = refs
    c = lax.axis_index("c")
    s = lax.axis_index("s")
    wid = s * NC + c
    nsup = KCH // nb

    zcp = pltpu.async_copy(zrow_hbm, acc_sh.at[pl.ds(s * RPT, RPT)], zsem)
    pltpu.sync_copy(src_hbm.at[wid], sidx_v)
    pltpu.sync_copy(dst_hbm.at[wid], didx_v)
    zcp.wait()
    plsc.subcore_barrier()

    def fire_gathers(sup, st):
        for b in range(nb):
            pltpu.async_copy(y_hbm.at[sidx_v.at[sup * nb + b]],
                             rows_v.at[st, b], gsem.at[st])

    def wait_gathers(st):
        for b in range(nb):
            pltpu.make_async_copy(y_hbm.at[pl.ds(0, CHUNK)],
                                  rows_v.at[st, b], gsem.at[st]).wait()

    def fire_scatters(sup, st):
        for b in range(nb):
            pltpu.async_copy(rows_v.at[st, b],
                             acc_sh.at[didx_v.at[sup * nb + b]],
                             ssem.at[st], add=True)

    def wait_scatters(st):
        for b in range(nb):
            pltpu.make_async_copy(rows_v.at[st, 0],
                                  acc_sh.at[pl.ds(0, CHUNK)],
                                  ssem.at[st]).wait()

    fire_gathers(0, 0)

    def sup_body(sidx, _):
        cur = sidx % 2
        oth = 1 - cur

        @pl.when(sidx + 1 < nsup)
        def _():
            @pl.when(sidx >= 1)
            def _():
                wait_scatters(oth)
            fire_gathers(sidx + 1, oth)

        wait_gathers(cur)
        fire_scatters(sidx, cur)
        return 0
    lax.fori_loop(0, nsup, sup_body, 0)
    wait_scatters((nsup - 2) % 2)
    wait_scatters((nsup - 1) % 2)

    plsc.subcore_barrier()
    pltpu.sync_copy(acc_sh.at[pl.ds(s * RPT, RPT)],
                    acc_out.at[c, pl.ds(s * RPT, RPT)])


def _make_seg_sum(f, nb):
    mesh = plsc.VectorSubcoreMesh(core_axis_name="c", subcore_axis_name="s")
    out_type = [jax.ShapeDtypeStruct((NC, NACC, f), jnp.float32)]
    scratch = [
        pltpu.VMEM((KCH, CHUNK), jnp.int32),
        pltpu.VMEM((KCH, CHUNK), jnp.int32),
        pltpu.VMEM((2, nb, CHUNK, f), jnp.float32),
        pltpu.VMEM_SHARED((NACC, f), jnp.float32),
        pltpu.SemaphoreType.DMA((2,)),
        pltpu.SemaphoreType.DMA((2,)),
        pltpu.SemaphoreType.DMA,
    ]
    return pl.kernel(
        functools.partial(_seg_sum_body, f, nb),
        out_type=out_type,
        mesh=mesh,
        scratch_types=scratch,
        compiler_params=pltpu.CompilerParams(
            needs_layout_passes=False, use_tc_tiling_on_sc=False),
    )


def _cnt_body(dst_hbm, cnt_out, didx_v, cnt_l):
    c = lax.axis_index("c")
    s = lax.axis_index("s")
    wid = s * NC + c
    pltpu.sync_copy(dst_hbm.at[wid], didx_v)

    def zero_body(i, _):
        cnt_l[pl.ds(i * 16, 16)] = jnp.zeros((16,), jnp.float32)
        return 0
    lax.fori_loop(0, NACC // 16, zero_body, 0)
    ones16 = jnp.full((16,), 1.0, jnp.float32)

    def cnt_body(j, _):
        def inner(o, _):
            idx16 = didx_v[j, pl.ds(o * 16, 16)]
            plsc.addupdate_scatter(cnt_l, [idx16], ones16,
                                   mask=jnp.ones((16,), jnp.bool_))
            return 0
        lax.fori_loop(0, CHUNK // 16, inner, 0)
        return 0
    lax.fori_loop(0, KCH, cnt_body, 0)
    pltpu.sync_copy(cnt_l, cnt_out.at[wid])


def _make_cnt():
    mesh = plsc.VectorSubcoreMesh(core_axis_name="c", subcore_axis_name="s")
    return pl.kernel(
        _cnt_body,
        out_type=jax.ShapeDtypeStruct((NW, NACC), jnp.float32),
        mesh=mesh,
        scratch_types=[
            pltpu.VMEM((KCH, CHUNK), jnp.int32),
            pltpu.VMEM((NACC,), jnp.float32),
        ],
        compiler_params=pltpu.CompilerParams(
            needs_layout_passes=False, use_tc_tiling_on_sc=False),
    )


def _tc1_body(x_ref, wl_ref, wr_ref, b_ref, y_ref, xr_ref):
    x = x_ref[...]
    y_ref[...] = jnp.dot(x, wl_ref[...], preferred_element_type=jnp.float32)
    xr_ref[...] = (
        jnp.dot(x, wr_ref[...], preferred_element_type=jnp.float32)
        + b_ref[...][None, :]
    )


def _tc2_body(acc_ref, cnt_ref, xr_ref, wl_ref, wr_ref, b_ref,
              y2_ref, xr2_ref, inv_ref):
    cnt = jnp.sum(cnt_ref[...], axis=0)
    inv = 1.0 / jnp.clip(cnt, 1.0, None)
    inv_ref[...] = inv[:, None]
    agg = acc_ref[0, :NN, :] + acc_ref[1, :NN, :]
    h1 = jnp.maximum(agg * inv[:NN, None] + xr_ref[...], 0.0)
    y2_ref[...] = jnp.dot(h1, wl_ref[...], preferred_element_type=jnp.float32)
    xr2_ref[...] = (
        jnp.dot(h1, wr_ref[...], preferred_element_type=jnp.float32)
        + b_ref[...][None, :]
    )


def _tc3_body(acc_ref, inv_ref, xr_ref, wfc_ref, bfc_ref, cost_ref, h_ref):
    agg = acc_ref[0, :NN, :] + acc_ref[1, :NN, :]
    h2 = jnp.maximum(agg * inv_ref[:NN, :] + xr_ref[...], 0.0)
    h_ref[...] = h2
    cost_ref[...] = (
        jnp.dot(h2, wfc_ref[...], preferred_element_type=jnp.float32)
        + bfc_ref[...][None, :]
    )


def kernel(x, edge_index, W1l, W1r, b1, W2l, W2r, b2, Wfc, bfc):
    src = edge_index[0]
    dst = edge_index[1]
    pad = EPAD - EE
    src_p = jnp.concatenate([src, jnp.zeros((pad,), jnp.int32)])
    dst_p = jnp.concatenate([dst, jnp.full((pad,), NN, jnp.int32)])
    src3 = src_p.reshape(NW, KCH, CHUNK)
    dst3 = dst_p.reshape(NW, KCH, CHUNK)
    zrow = jnp.zeros((RPT, HH), jnp.float32)
    zrow2 = jnp.zeros((RPT, OO), jnp.float32)

    cnt = _make_cnt()(dst3)

    y1, xr1 = pl.pallas_call(
        _tc1_body,
        out_shape=[jax.ShapeDtypeStruct((NN, HH), jnp.float32),
                   jax.ShapeDtypeStruct((NN, HH), jnp.float32)],
    )(x, W1l, W1r, b1)

    acc1, = _make_seg_sum(HH, 4)(y1, src3, dst3, zrow)

    y2, xr2, inv = pl.pallas_call(
        _tc2_body,
        out_shape=[jax.ShapeDtypeStruct((NN, OO), jnp.float32),
                   jax.ShapeDtypeStruct((NN, OO), jnp.float32),
                   jax.ShapeDtypeStruct((NACC, 1), jnp.float32)],
    )(acc1, cnt, xr1, W2l, W2r, b2)

    acc2, = _make_seg_sum(OO, 8)(y2, src3, dst3, zrow2)

    cost, h = pl.pallas_call(
        _tc3_body,
        out_shape=[jax.ShapeDtypeStruct((NN, 1), jnp.float32),
                   jax.ShapeDtypeStruct((NN, OO), jnp.float32)],
    )(acc2, inv, xr2, Wfc, bfc)

    return cost, h


# speedup vs baseline: 1.0089x; 1.0089x over previous
import functools

import jax
import jax.numpy as jnp
from jax import lax
from jax.experimental import pallas as pl
from jax.experimental.pallas import tpu as pltpu
from jax.experimental.pallas import tpu_sc as plsc

NN = 10000
EE = 320000
DD = 128
HH = 64
OO = 32

NC = 2
NS = 16
NW = NC * NS
CHUNK = 128
KCH = 80
EPAD = NW * KCH * CHUNK
NACC = 10112
RPT = NACC // NS


def _seg_sum_body(f, nb, y_hbm, src_hbm, dst_hbm, zrow_hbm, *refs):
    acc_out, sidx_v, didx_v, rows_v, acc_sh, gsem, ssem, zsem = refs
    c = lax.axis_index("c")
    s = lax.axis_index("s")
    wid = s * NC + c
    nsup = KCH // nb

    zcp = pltpu.async_copy(zrow_hbm, acc_sh.at[pl.ds(s * RPT, RPT)], zsem)
    pltpu.sync_copy(src_hbm.at[wid], sidx_v)
    pltpu.sync_copy(dst_hbm.at[wid], didx_v)
    zcp.wait()
    plsc.subcore_barrier()

    def fire_gathers(sup, st):
        for b in range(nb):
            pltpu.async_copy(y_hbm.at[sidx_v.at[sup * nb + b]],
                             rows_v.at[st, b], gsem.at[st])

    def wait_gathers(st):
        for b in range(nb):
            pltpu.make_async_copy(y_hbm.at[pl.ds(0, CHUNK)],
                                  rows_v.at[st, b], gsem.at[st]).wait()

    def fire_scatters(sup, st):
        for b in range(nb):
            pltpu.async_copy(rows_v.at[st, b],
                             acc_sh.at[didx_v.at[sup * nb + b]],
                             ssem.at[st], add=True)

    def wait_scatters(st):
        for b in range(nb):
            pltpu.make_async_copy(rows_v.at[st, 0],
                                  acc_sh.at[pl.ds(0, CHUNK)],
                                  ssem.at[st]).wait()

    del nsup, fire_gathers, wait_gathers, fire_scatters, wait_scatters
    for b in range(nb):
        pltpu.async_copy(y_hbm.at[sidx_v.at[b]], rows_v.at[0, b],
                         gsem.at[b])

    def outer_body(g, _):
        for b in range(nb):
            j = g * nb + b
            pltpu.make_async_copy(y_hbm.at[pl.ds(0, CHUNK)],
                                  rows_v.at[0, b], gsem.at[b]).wait()
            pltpu.sync_copy(rows_v.at[0, b], acc_sh.at[didx_v.at[j]],
                            add=True)

            @pl.when(j + nb < KCH)
            def _():
                pltpu.async_copy(y_hbm.at[sidx_v.at[j + nb]],
                                 rows_v.at[0, b], gsem.at[b])
        return 0
    lax.fori_loop(0, KCH // nb, outer_body, 0)

    plsc.subcore_barrier()
    pltpu.sync_copy(acc_sh.at[pl.ds(s * RPT, RPT)],
                    acc_out.at[c, pl.ds(s * RPT, RPT)])


def _make_seg_sum(f, nb):
    mesh = plsc.VectorSubcoreMesh(core_axis_name="c", subcore_axis_name="s")
    out_type = [jax.ShapeDtypeStruct((NC, NACC, f), jnp.float32)]
    scratch = [
        pltpu.VMEM((KCH, CHUNK), jnp.int32),
        pltpu.VMEM((KCH, CHUNK), jnp.int32),
        pltpu.VMEM((2, nb, CHUNK, f), jnp.float32),
        pltpu.VMEM_SHARED((NACC, f), jnp.float32),
        pltpu.SemaphoreType.DMA((nb,)),
        pltpu.SemaphoreType.DMA((nb,)),
        pltpu.SemaphoreType.DMA,
    ]
    return pl.kernel(
        functools.partial(_seg_sum_body, f, nb),
        out_type=out_type,
        mesh=mesh,
        scratch_types=scratch,
        compiler_params=pltpu.CompilerParams(
            needs_layout_passes=False, use_tc_tiling_on_sc=False),
    )


def _cnt_body(dst_hbm, cnt_out, didx_v, cnt_l):
    c = lax.axis_index("c")
    s = lax.axis_index("s")
    wid = s * NC + c
    pltpu.sync_copy(dst_hbm.at[wid], didx_v)

    def zero_body(i, _):
        cnt_l[pl.ds(i * 16, 16)] = jnp.zeros((16,), jnp.float32)
        return 0
    lax.fori_loop(0, NACC // 16, zero_body, 0)
    ones16 = jnp.full((16,), 1.0, jnp.float32)

    def cnt_body(j, _):
        def inner(o, _):
            idx16 = didx_v[j, pl.ds(o * 16, 16)]
            plsc.addupdate_scatter(cnt_l, [idx16], ones16,
                                   mask=jnp.ones((16,), jnp.bool_))
            return 0
        lax.fori_loop(0, CHUNK // 16, inner, 0)
        return 0
    lax.fori_loop(0, KCH, cnt_body, 0)
    pltpu.sync_copy(cnt_l, cnt_out.at[wid])


def _make_cnt():
    mesh = plsc.VectorSubcoreMesh(core_axis_name="c", subcore_axis_name="s")
    return pl.kernel(
        _cnt_body,
        out_type=jax.ShapeDtypeStruct((NW, NACC), jnp.float32),
        mesh=mesh,
        scratch_types=[
            pltpu.VMEM((KCH, CHUNK), jnp.int32),
            pltpu.VMEM((NACC,), jnp.float32),
        ],
        compiler_params=pltpu.CompilerParams(
            needs_layout_passes=False, use_tc_tiling_on_sc=False),
    )


def _tc1_body(x_ref, wl_ref, wr_ref, b_ref, y_ref, xr_ref):
    x = x_ref[...]
    y_ref[...] = jnp.dot(x, wl_ref[...], preferred_element_type=jnp.float32)
    xr_ref[...] = (
        jnp.dot(x, wr_ref[...], preferred_element_type=jnp.float32)
        + b_ref[...][None, :]
    )


def _tc2_body(acc_ref, cnt_ref, xr_ref, wl_ref, wr_ref, b_ref,
              y2_ref, xr2_ref, inv_ref):
    cnt = jnp.sum(cnt_ref[...], axis=0)
    inv = 1.0 / jnp.clip(cnt, 1.0, None)
    inv_ref[...] = inv[:, None]
    agg = acc_ref[0, :NN, :] + acc_ref[1, :NN, :]
    h1 = jnp.maximum(agg * inv[:NN, None] + xr_ref[...], 0.0)
    y2_ref[...] = jnp.dot(h1, wl_ref[...], preferred_element_type=jnp.float32)
    xr2_ref[...] = (
        jnp.dot(h1, wr_ref[...], preferred_element_type=jnp.float32)
        + b_ref[...][None, :]
    )


def _tc3_body(acc_ref, inv_ref, xr_ref, wfc_ref, bfc_ref, cost_ref, h_ref):
    agg = acc_ref[0, :NN, :] + acc_ref[1, :NN, :]
    h2 = jnp.maximum(agg * inv_ref[:NN, :] + xr_ref[...], 0.0)
    h_ref[...] = h2
    cost_ref[...] = (
        jnp.dot(h2, wfc_ref[...], preferred_element_type=jnp.float32)
        + bfc_ref[...][None, :]
    )


def kernel(x, edge_index, W1l, W1r, b1, W2l, W2r, b2, Wfc, bfc):
    src = edge_index[0]
    dst = edge_index[1]
    pad = EPAD - EE
    src_p = jnp.concatenate([src, jnp.zeros((pad,), jnp.int32)])
    dst_p = jnp.concatenate([dst, jnp.full((pad,), NN, jnp.int32)])
    src3 = src_p.reshape(NW, KCH, CHUNK)
    dst3 = dst_p.reshape(NW, KCH, CHUNK)
    zrow = jnp.zeros((RPT, HH), jnp.float32)
    zrow2 = jnp.zeros((RPT, OO), jnp.float32)

    cnt = _make_cnt()(dst3)

    y1, xr1 = pl.pallas_call(
        _tc1_body,
        out_shape=[jax.ShapeDtypeStruct((NN, HH), jnp.float32),
                   jax.ShapeDtypeStruct((NN, HH), jnp.float32)],
    )(x, W1l, W1r, b1)

    acc1, = _make_seg_sum(HH, 4)(y1, src3, dst3, zrow)

    y2, xr2, inv = pl.pallas_call(
        _tc2_body,
        out_shape=[jax.ShapeDtypeStruct((NN, OO), jnp.float32),
                   jax.ShapeDtypeStruct((NN, OO), jnp.float32),
                   jax.ShapeDtypeStruct((NACC, 1), jnp.float32)],
    )(acc1, cnt, xr1, W2l, W2r, b2)

    acc2, = _make_seg_sum(OO, 8)(y2, src3, dst3, zrow2)

    cost, h = pl.pallas_call(
        _tc3_body,
        out_shape=[jax.ShapeDtypeStruct((NN, 1), jnp.float32),
                   jax.ShapeDtypeStruct((NN, OO), jnp.float32)],
    )(acc2, inv, xr2, Wfc, bfc)

    return cost, h


# speedup vs baseline: 2.1591x; 2.1400x over previous
import functools

import jax
import jax.numpy as jnp
from jax import lax
from jax.experimental import pallas as pl
from jax.experimental.pallas import tpu as pltpu
from jax.experimental.pallas import tpu_sc as plsc

NN = 10000
EE = 320000
DD = 128
HH = 64
OO = 32

NC = 2
NS = 16
NW = NC * NS
CHUNK = 128
KCH = 80
EPAD = NW * KCH * CHUNK
NACC = 10112
RPT = NACC // NS


def _seg_sum_body(f, nb, y_hbm, src_hbm, dst_hbm, zrow_hbm, *refs):
    acc_out, sidx_v, didx_v, rows_v, y_sh, acc_sh, gsem, ssem, zsem = refs
    c = lax.axis_index("c")
    s = lax.axis_index("s")
    wid = s * NC + c

    ycp = pltpu.async_copy(y_hbm.at[pl.ds(s * (NN // NS), NN // NS)],
                           y_sh.at[pl.ds(s * (NN // NS), NN // NS)], zsem)
    zcp = pltpu.async_copy(zrow_hbm, acc_sh.at[pl.ds(s * RPT, RPT)], zsem)
    pltpu.sync_copy(src_hbm.at[wid], sidx_v)
    pltpu.sync_copy(dst_hbm.at[wid], didx_v)
    ycp.wait()
    zcp.wait()
    plsc.subcore_barrier()

    def fire_gathers(sup, st):
        for b in range(nb):
            pltpu.async_copy(y_hbm.at[sidx_v.at[sup * nb + b]],
                             rows_v.at[st, b], gsem.at[st])

    def wait_gathers(st):
        for b in range(nb):
            pltpu.make_async_copy(y_hbm.at[pl.ds(0, CHUNK)],
                                  rows_v.at[st, b], gsem.at[st]).wait()

    def fire_scatters(sup, st):
        for b in range(nb):
            pltpu.async_copy(rows_v.at[st, b],
                             acc_sh.at[didx_v.at[sup * nb + b]],
                             ssem.at[st], add=True)

    def wait_scatters(st):
        for b in range(nb):
            pltpu.make_async_copy(rows_v.at[st, 0],
                                  acc_sh.at[pl.ds(0, CHUNK)],
                                  ssem.at[st]).wait()

    del fire_gathers, wait_gathers, fire_scatters, wait_scatters
    for b in range(nb):
        pltpu.async_copy(y_sh.at[sidx_v.at[b]], rows_v.at[b],
                         gsem.at[b])

    def outer_body(g, _):
        for b in range(nb):
            j = g * nb + b
            pltpu.make_async_copy(y_sh.at[pl.ds(0, CHUNK)],
                                  rows_v.at[b], gsem.at[b]).wait()
            pltpu.sync_copy(rows_v.at[b], acc_sh.at[didx_v.at[j]],
                            add=True)

            @pl.when(j + nb < KCH)
            def _():
                pltpu.async_copy(y_sh.at[sidx_v.at[j + nb]],
                                 rows_v.at[b], gsem.at[b])
        return 0
    lax.fori_loop(0, KCH // nb, outer_body, 0)

    plsc.subcore_barrier()
    pltpu.sync_copy(acc_sh.at[pl.ds(s * RPT, RPT)],
                    acc_out.at[c, pl.ds(s * RPT, RPT)])


def _make_seg_sum(f, nb):
    mesh = plsc.VectorSubcoreMesh(core_axis_name="c", subcore_axis_name="s")
    out_type = [jax.ShapeDtypeStruct((NC, NACC, f), jnp.float32)]
    scratch = [
        pltpu.VMEM((KCH, CHUNK), jnp.int32),
        pltpu.VMEM((KCH, CHUNK), jnp.int32),
        pltpu.VMEM((nb, CHUNK, f), jnp.float32),
        pltpu.VMEM_SHARED((NN, f), jnp.float32),
        pltpu.VMEM_SHARED((NACC, f), jnp.float32),
        pltpu.SemaphoreType.DMA((nb,)),
        pltpu.SemaphoreType.DMA((nb,)),
        pltpu.SemaphoreType.DMA,
    ]
    return pl.kernel(
        functools.partial(_seg_sum_body, f, nb),
        out_type=out_type,
        mesh=mesh,
        scratch_types=scratch,
        compiler_params=pltpu.CompilerParams(
            needs_layout_passes=False, use_tc_tiling_on_sc=False),
    )


def _cnt_body(dst_hbm, cnt_out, didx_v, cnt_l):
    c = lax.axis_index("c")
    s = lax.axis_index("s")
    wid = s * NC + c
    pltpu.sync_copy(dst_hbm.at[wid], didx_v)

    def zero_body(i, _):
        cnt_l[pl.ds(i * 16, 16)] = jnp.zeros((16,), jnp.float32)
        return 0
    lax.fori_loop(0, NACC // 16, zero_body, 0)
    ones16 = jnp.full((16,), 1.0, jnp.float32)

    def cnt_body(j, _):
        def inner(o, _):
            idx16 = didx_v[j, pl.ds(o * 16, 16)]
            plsc.addupdate_scatter(cnt_l, [idx16], ones16,
                                   mask=jnp.ones((16,), jnp.bool_))
            return 0
        lax.fori_loop(0, CHUNK // 16, inner, 0)
        return 0
    lax.fori_loop(0, KCH, cnt_body, 0)
    pltpu.sync_copy(cnt_l, cnt_out.at[wid])


def _make_cnt():
    mesh = plsc.VectorSubcoreMesh(core_axis_name="c", subcore_axis_name="s")
    return pl.kernel(
        _cnt_body,
        out_type=jax.ShapeDtypeStruct((NW, NACC), jnp.float32),
        mesh=mesh,
        scratch_types=[
            pltpu.VMEM((KCH, CHUNK), jnp.int32),
            pltpu.VMEM((NACC,), jnp.float32),
        ],
        compiler_params=pltpu.CompilerParams(
            needs_layout_passes=False, use_tc_tiling_on_sc=False),
    )


def _tc1_body(x_ref, wl_ref, wr_ref, b_ref, y_ref, xr_ref):
    x = x_ref[...]
    y_ref[...] = jnp.dot(x, wl_ref[...], preferred_element_type=jnp.float32)
    xr_ref[...] = (
        jnp.dot(x, wr_ref[...], preferred_element_type=jnp.float32)
        + b_ref[...][None, :]
    )


def _tc2_body(acc_ref, cnt_ref, xr_ref, wl_ref, wr_ref, b_ref,
              y2_ref, xr2_ref, inv_ref):
    cnt = jnp.sum(cnt_ref[...], axis=0)
    inv = 1.0 / jnp.clip(cnt, 1.0, None)
    inv_ref[...] = inv[:, None]
    agg = acc_ref[0, :NN, :] + acc_ref[1, :NN, :]
    h1 = jnp.maximum(agg * inv[:NN, None] + xr_ref[...], 0.0)
    y2_ref[...] = jnp.dot(h1, wl_ref[...], preferred_element_type=jnp.float32)
    xr2_ref[...] = (
        jnp.dot(h1, wr_ref[...], preferred_element_type=jnp.float32)
        + b_ref[...][None, :]
    )


def _tc3_body(acc_ref, inv_ref, xr_ref, wfc_ref, bfc_ref, cost_ref, h_ref):
    agg = acc_ref[0, :NN, :] + acc_ref[1, :NN, :]
    h2 = jnp.maximum(agg * inv_ref[:NN, :] + xr_ref[...], 0.0)
    h_ref[...] = h2
    cost_ref[...] = (
        jnp.dot(h2, wfc_ref[...], preferred_element_type=jnp.float32)
        + bfc_ref[...][None, :]
    )


def kernel(x, edge_index, W1l, W1r, b1, W2l, W2r, b2, Wfc, bfc):
    src = edge_index[0]
    dst = edge_index[1]
    pad = EPAD - EE
    src_p = jnp.concatenate([src, jnp.zeros((pad,), jnp.int32)])
    dst_p = jnp.concatenate([dst, jnp.full((pad,), NN, jnp.int32)])
    src3 = src_p.reshape(NW, KCH, CHUNK)
    dst3 = dst_p.reshape(NW, KCH, CHUNK)
    zrow = jnp.zeros((RPT, HH), jnp.float32)
    zrow2 = jnp.zeros((RPT, OO), jnp.float32)

    cnt = _make_cnt()(dst3)

    y1, xr1 = pl.pallas_call(
        _tc1_body,
        out_shape=[jax.ShapeDtypeStruct((NN, HH), jnp.float32),
                   jax.ShapeDtypeStruct((NN, HH), jnp.float32)],
    )(x, W1l, W1r, b1)

    acc1, = _make_seg_sum(HH, 2)(y1, src3, dst3, zrow)

    y2, xr2, inv = pl.pallas_call(
        _tc2_body,
        out_shape=[jax.ShapeDtypeStruct((NN, OO), jnp.float32),
                   jax.ShapeDtypeStruct((NN, OO), jnp.float32),
                   jax.ShapeDtypeStruct((NACC, 1), jnp.float32)],
    )(acc1, cnt, xr1, W2l, W2r, b2)

    acc2, = _make_seg_sum(OO, 2)(y2, src3, dst3, zrow2)

    cost, h = pl.pallas_call(
        _tc3_body,
        out_shape=[jax.ShapeDtypeStruct((NN, 1), jnp.float32),
                   jax.ShapeDtypeStruct((NN, OO), jnp.float32)],
    )(acc2, inv, xr2, Wfc, bfc)

    return cost, h
